# baseline (speedup 1.0000x reference)
import functools

import jax
import jax.numpy as jnp
from jax import lax
from jax.experimental import pallas as pl
from jax.experimental.pallas import tpu as pltpu
from jax.experimental.pallas import tpu_sc as plsc


def _emb_lookup(table, ids3, N, D, DP, NC, NW, n_ch, CH):
    mesh = plsc.VectorSubcoreMesh(core_axis_name="c", subcore_axis_name="s")

    @functools.partial(
        pl.kernel,
        mesh=mesh,
        out_type=jax.ShapeDtypeStruct((N, DP), table.dtype),
        compiler_params=pltpu.CompilerParams(use_tc_tiling_on_sc=False),
        scratch_types=[
            pltpu.VMEM((n_ch, CH), jnp.int32),
            pltpu.VMEM((CH, DP), table.dtype),
            pltpu.SemaphoreType.DMA,
            pltpu.SemaphoreType.DMA,
        ],
    )
    def emb(table_hbm, ids_hbm, out_hbm, idx_v, buf, gsem, osem):
        per_w = n_ch * CH
        wid = lax.axis_index("s") * NC + lax.axis_index("c")
        base = wid * per_w
        pltpu.sync_copy(ids_hbm.at[wid], idx_v)

        @pl.loop(0, n_ch)
        def _body(c):
            pltpu.async_copy(table_hbm.at[idx_v.at[c]], buf, gsem).wait()
            pltpu.async_copy(
                buf, out_hbm.at[pl.ds(base + c * CH, CH)], osem
            ).wait()

    return emb(table, ids3)


def kernel(table, _input_token_ids):
    V, D = table.shape
    Bt, Lt = _input_token_ids.shape
    N = Bt * Lt
    DP = 304
    table = jnp.pad(table, ((0, 0), (0, DP - D)))
    info = plsc.get_sparse_core_info()
    NC = info.num_cores
    NW = NC * info.num_subcores
    CH = 64
    assert N % (NW * CH) == 0
    n_ch = N // (NW * CH)
    ids3 = _input_token_ids.reshape(NW, n_ch, CH)
    out = _emb_lookup(table, ids3, N, D, DP, NC, NW, n_ch, CH)
    return out[:, :D].reshape(Bt, Lt, D)


# speedup vs baseline: 1.0407x; 1.0407x over previous
import functools

import jax
import jax.numpy as jnp
from jax import lax
from jax.experimental import pallas as pl
from jax.experimental.pallas import tpu as pltpu
from jax.experimental.pallas import tpu_sc as plsc


def _emb_lookup(table, ids3, N, D, DP, NC, NW, n_ch, CH, K):
    mesh = plsc.VectorSubcoreMesh(core_axis_name="c", subcore_axis_name="s")

    @functools.partial(
        pl.kernel,
        mesh=mesh,
        out_type=jax.ShapeDtypeStruct((N, DP), table.dtype),
        compiler_params=pltpu.CompilerParams(use_tc_tiling_on_sc=False),
        scratch_types=(
            [pltpu.VMEM((n_ch, CH), jnp.int32)]
            + [pltpu.VMEM((CH, DP), table.dtype) for _ in range(K)]
            + [pltpu.SemaphoreType.DMA for _ in range(2 * K)]
        ),
    )
    def emb(table_hbm, ids_hbm, out_hbm, idx_v, *rest):
        bufs = rest[:K]
        gsem = rest[K : 2 * K]
        osem = rest[2 * K : 3 * K]
        per_w = n_ch * CH
        wid = lax.axis_index("s") * NC + lax.axis_index("c")
        base = wid * per_w
        pltpu.sync_copy(ids_hbm.at[wid], idx_v)
        for b in range(K - 1):
            pltpu.async_copy(table_hbm.at[idx_v.at[b]], bufs[b], gsem[b])

        @pl.loop(0, n_ch // K)
        def _outer(g):
            c0 = g * K
            for b in range(K):
                c = c0 + b
                pltpu.make_async_copy(
                    table_hbm.at[idx_v.at[c]], bufs[b], gsem[b]
                ).wait()
                pltpu.async_copy(
                    bufs[b], out_hbm.at[pl.ds(base + c * CH, CH)], osem[b]
                )
                nb = (b + K - 1) % K
                nxt = c + K - 1

                @pl.when(nxt < n_ch)
                def _():
                    @pl.when(c >= 1)
                    def _():
                        pltpu.make_async_copy(
                            bufs[nb],
                            out_hbm.at[pl.ds(base + (c - 1) * CH, CH)],
                            osem[nb],
                        ).wait()

                    pltpu.async_copy(
                        table_hbm.at[idx_v.at[nxt]], bufs[nb], gsem[nb]
                    )

        for b in range(K):
            c = n_ch - K + b
            pltpu.make_async_copy(
                bufs[b], out_hbm.at[pl.ds(base + c * CH, CH)], osem[b]
            ).wait()

    return emb(table, ids3)


def kernel(table, _input_token_ids):
    V, D = table.shape
    Bt, Lt = _input_token_ids.shape
    N = Bt * Lt
    DP = 304
    table = jnp.pad(table, ((0, 0), (0, DP - D)))
    info = plsc.get_sparse_core_info()
    NC = info.num_cores
    NW = NC * info.num_subcores
    CH = 64
    K = 4
    assert N % (NW * CH) == 0
    n_ch = N // (NW * CH)
    assert n_ch % K == 0
    ids3 = _input_token_ids.reshape(NW, n_ch, CH)
    out = _emb_lookup(table, ids3, N, D, DP, NC, NW, n_ch, CH, K)
    return out[:, :D].reshape(Bt, Lt, D)
